# baseline (speedup 1.0000x reference)
import functools

import jax
import jax.numpy as jnp
from jax import lax
from jax.experimental import pallas as pl
from jax.experimental.pallas import tpu as pltpu
from jax.experimental.pallas import tpu_sc as plsc

EMB = 128
MAX_LEN = 512
BATCH = 16384
HIST = 200

NUM_CORES = 2
NUM_SUBCORES = 16
NUM_WORKERS = NUM_CORES * NUM_SUBCORES
LANES = 16

ROWS_PER_CHUNK = 2
ENT = ROWS_PER_CHUNK * HIST
NVEC = ENT // LANES
CHUNKS_PER_WORKER = BATCH // (NUM_WORKERS * ROWS_PER_CHUNK)
GATHER_GROUP = 80
NGROUPS = ENT // GATHER_GROUP


def _vgather(v, idx):
    dnums = lax.GatherDimensionNumbers(
        offset_dims=(), collapsed_slice_dims=(0,), start_index_map=(0,))
    return lax.gather(v, idx[:, None], dnums, (1,),
                      mode=lax.GatherScatterMode.PROMISE_IN_BOUNDS)


def _make_sc_kernel():
    mesh = plsc.VectorSubcoreMesh(core_axis_name="c", subcore_axis_name="s")

    @functools.partial(
        pl.kernel,
        mesh=mesh,
        out_type=jax.ShapeDtypeStruct((BATCH * HIST, EMB), jnp.float32),
        scratch_types=[
            pltpu.VMEM((ENT,), jnp.int32),
            pltpu.VMEM((ENT, EMB), jnp.float32),
            pltpu.SemaphoreType.DMA,
        ],
    )
    def sc_embed(vo_hbm, pe_hbm, out_hbm, idx_v, rows_v, sem):
        wid = lax.axis_index("s") * NUM_CORES + lax.axis_index("c")
        zeros16 = jnp.zeros((LANES,), jnp.int32)

        def chunk_body(c, carry):
            base = (wid * CHUNKS_PER_WORKER + c) * ENT
            pltpu.sync_copy(vo_hbm.at[pl.ds(base, ENT)], idx_v)
            f0 = _vgather(idx_v[pl.ds(0, LANES)], zeros16)
            f1 = _vgather(idx_v[pl.ds(HIST, LANES)], zeros16)
            for i in range(NVEC):
                lo = i * LANES
                v = idx_v[pl.ds(lo, LANES)]
                if lo + LANES <= HIST:
                    first = f0
                elif lo >= HIST:
                    first = f1
                else:
                    ent = lax.iota(jnp.int32, LANES) + lo
                    first = jnp.where(ent < HIST, f0, f1)
                idx_v[pl.ds(lo, LANES)] = jnp.clip(v - first, 0, MAX_LEN - 1)
            copies = []
            for g in range(NGROUPS):
                o = g * GATHER_GROUP
                copies.append(
                    pltpu.async_copy(
                        pe_hbm.at[idx_v.at[pl.ds(o, GATHER_GROUP)]],
                        rows_v.at[pl.ds(o, GATHER_GROUP)],
                        sem,
                    )
                )
            for cp in copies:
                cp.wait()
            pltpu.sync_copy(rows_v, out_hbm.at[pl.ds(base, ENT)])
            return carry

        lax.fori_loop(0, CHUNKS_PER_WORKER, chunk_body, 0)

    return sc_embed


_SC_EMBED = _make_sc_kernel()


@jax.jit
def kernel(visit_orders, pe):
    vo_flat = visit_orders.astype(jnp.int32).reshape(BATCH * HIST)
    out = _SC_EMBED(vo_flat, pe)
    return out.reshape(BATCH, HIST, EMB)
